# baseline (speedup 1.0000x reference)
import functools

import jax
import jax.numpy as jnp
from jax import lax
from jax.experimental import pallas as pl
from jax.experimental.pallas import tpu as pltpu
from jax.experimental.pallas import tpu_sc as plsc

NUM_CLASSES = 1000000
EMBED_DIM = 32
BATCH = 16384
FIELDS = 26

NC = 2
NS = 16
NW = NC * NS
LANES = 16

N_FULL = NUM_CLASSES // 128
TAIL = NUM_CLASSES - N_FULL * 128
A_BASE = N_FULL // NW
A_EXTRA = N_FULL - A_BASE * NW

TOTAL = BATCH * FIELDS
PER_W = TOTAL // NW
IDX_B = 128
GPC = 2
CHUNK = IDX_B * GPC
N_CHUNKS = PER_W // CHUNK
N_GATHER = PER_W // IDX_B
NBUF = 4
N_MAIN = N_CHUNKS // NBUF - 1

assert PER_W * NW == TOTAL
assert CHUNK * N_CHUNKS == PER_W
assert N_CHUNKS % NBUF == 0


def _make_transpose():
    mesh = plsc.VectorSubcoreMesh(core_axis_name="c", subcore_axis_name="s")

    @functools.partial(
        pl.kernel,
        mesh=mesh,
        out_type=jax.ShapeDtypeStruct((NUM_CLASSES // 4, 128), jnp.float32),
        scratch_types=[
            pltpu.VMEM((2, 32, 128), jnp.float32),
            pltpu.VMEM((2, 32, 128), jnp.float32),
        ]
        + [pltpu.SemaphoreType.DMA] * 4,
        compiler_params=pltpu.CompilerParams(
            use_tc_tiling_on_sc=True, needs_layout_passes=False
        ),
    )
    def transpose_kernel(wt_hbm, tail_hbm, w128_hbm, buf_in, buf_out, *sems):
        isem = sems[:2]
        osem = sems[2:]
        wid = lax.axis_index("s") * NC + lax.axis_index("c")
        start = wid * A_BASE + jnp.minimum(wid, A_EXTRA)
        end = start + A_BASE + jnp.where(wid < A_EXTRA, 1, 0)

        lane = lax.iota(jnp.int32, LANES)

        def start_in(vb, p):
            pltpu.async_copy(
                wt_hbm.at[:, pl.ds(vb * 128, 128)],
                buf_in.at[p],
                isem[p],
            )

        def wait_in(p):
            pltpu.make_async_copy(
                wt_hbm.at[:, pl.ds(0, 128)],
                buf_in.at[p],
                isem[p],
            ).wait()

        def start_out(vb, p):
            pltpu.async_copy(
                buf_out.at[p],
                w128_hbm.at[pl.ds(vb * 32, 32)],
                osem[p],
            )

        def wait_out(p):
            pltpu.make_async_copy(
                buf_out.at[p],
                w128_hbm.at[pl.ds(0, 32)],
                osem[p],
            ).wait()

        def transpose_block(p):
            for jj in range(32):
                for g in range(8):
                    row = lane + 16 * (g % 2)
                    col = jnp.full((LANES,), 4 * jj + g // 2, jnp.int32)
                    v = plsc.load_gather(buf_in.at[p], [row, col])
                    buf_out.at[p][jj, pl.ds(g * 16, LANES)] = v

        start_in(start, 0)

        def body2(i, _):
            vb = start + i * 2
            @pl.when(vb < end)
            def _():
                wait_in(0)

                @pl.when(vb + 1 < end)
                def _():
                    start_in(vb + 1, 1)

                transpose_block(0)
                start_out(vb, 0)

            @pl.when(vb + 1 < end)
            def _():
                wait_in(1)

                @pl.when(vb + 2 < end)
                def _():
                    start_in(vb + 2, 0)

                transpose_block(1)
                start_out(vb + 1, 1)

            @pl.when(vb < end)
            def _():
                wait_out(0)

            @pl.when(vb + 1 < end)
            def _():
                wait_out(1)

            return 0

        n_pairs = (A_BASE + 2) // 2
        lax.fori_loop(0, n_pairs, body2, 0)

        @pl.when(wid == 0)
        def _():
            pltpu.async_copy(
                tail_hbm,
                buf_in.at[0].at[pl.ds(0, 16)],
                isem[0],
            ).wait()
            pltpu.async_copy(
                buf_in.at[0].at[pl.ds(0, 16)],
                w128_hbm.at[pl.ds(N_FULL * 32, 16)],
                osem[0],
            ).wait()

    return transpose_kernel


def _make_gather():
    mesh = plsc.VectorSubcoreMesh(core_axis_name="c", subcore_axis_name="s")

    @functools.partial(
        pl.kernel,
        mesh=mesh,
        out_type=jax.ShapeDtypeStruct((TOTAL, EMBED_DIM), jnp.float32),
        scratch_types=[
            pltpu.VMEM((N_GATHER, IDX_B), jnp.int32),
            pltpu.VMEM((NBUF, CHUNK, EMBED_DIM), jnp.float32),
        ]
        + [pltpu.SemaphoreType.DMA] * (2 * NBUF),
        compiler_params=pltpu.CompilerParams(use_tc_tiling_on_sc=False),
    )
    def gather_kernel(table_hbm, idx_hbm, out_hbm, idx_v, rows_v, *sems):
        gsem = sems[:NBUF]
        osem = sems[NBUF:]
        wid = lax.axis_index("s") * NC + lax.axis_index("c")
        pltpu.sync_copy(idx_hbm.at[wid], idx_v)

        def start_gather(c, b):
            for j in range(GPC):
                pltpu.async_copy(
                    table_hbm.at[idx_v.at[c * GPC + j]],
                    rows_v.at[b].at[pl.ds(j * IDX_B, IDX_B)],
                    gsem[b],
                )

        def wait_gather(b):
            for j in range(GPC):
                pltpu.make_async_copy(
                    table_hbm.at[idx_v.at[j]],
                    rows_v.at[b].at[pl.ds(j * IDX_B, IDX_B)],
                    gsem[b],
                ).wait()

        def start_out(c, b):
            pltpu.async_copy(
                rows_v.at[b],
                out_hbm.at[pl.ds(wid * PER_W + c * CHUNK, CHUNK)],
                osem[b],
            )

        def wait_out(b):
            pltpu.make_async_copy(
                rows_v.at[b],
                out_hbm.at[pl.ds(wid * PER_W, CHUNK)],
                osem[b],
            ).wait()

        for b in range(NBUF):
            start_gather(jnp.int32(b), b)

        def body(g, _):
            for b in range(NBUF):
                c = g * NBUF + b
                wait_gather(b)
                start_out(c, b)
            for b in range(NBUF):
                c_next = (g + 1) * NBUF + b
                wait_out(b)
                start_gather(c_next, b)
            return 0

        lax.fori_loop(0, N_MAIN, body, 0)

        for b in range(NBUF):
            c = N_MAIN * NBUF + b
            wait_gather(b)
            start_out(jnp.int32(c), b)
        for b in range(NBUF):
            wait_out(b)

    return gather_kernel


_transpose = _make_transpose()
_gather = _make_gather()


def kernel(severity_ids, weight):
    tail = lax.slice(weight, (N_FULL * 128, 0), (NUM_CLASSES, EMBED_DIM))
    w128 = _transpose(weight.T, tail.reshape(16, 128))
    table = w128.reshape(NUM_CLASSES, EMBED_DIM)
    idx = severity_ids.reshape(NW, N_GATHER, IDX_B).astype(jnp.int32)
    out = _gather(table, idx)
    return out.reshape(BATCH, FIELDS, EMBED_DIM)
